# baseline (speedup 1.0000x reference)
import functools

import jax
import jax.numpy as jnp
from jax import lax
from jax.experimental import pallas as pl
from jax.experimental.pallas import tpu as pltpu
from jax.experimental.pallas import tpu_sc as plsc

N_NODES = 10000
N_EDGES = 320000
F = 128

NC, NS, L = 2, 16, 16
NW = NC * NS
EPW = N_EDGES // NW
NVR = EPW // L
LIST = EPW + L
NNP = 10240
RPT = NNP // NS
CB = 200
NFILL = EPW // CB
KSB = 8
ELIST = EPW + KSB * L

def _last(v):
    return lax.squeeze(lax.slice_in_dim(v, L - 1, L), (0,))


_SC_SUMS_KW = dict(
    out_type=(
        jax.ShapeDtypeStruct((NC, 16, F), jnp.float32),
        jax.ShapeDtypeStruct((NC, 16, F), jnp.float32),
    ),
    scratch_types=[
        pltpu.VMEM((EPW,), jnp.int32),
        pltpu.VMEM((LIST,), jnp.int32),
        pltpu.VMEM((LIST,), jnp.int32),
        pltpu.VMEM((L, F), jnp.float32),
        pltpu.VMEM((L, F), jnp.float32),
        pltpu.VMEM_SHARED((16, F), jnp.float32),
        pltpu.VMEM_SHARED((16, F), jnp.float32),
        pltpu.SemaphoreType.DMA,
    ],
)


def _sc_sums_body(dst_hbm, feats_hbm, zc_hbm, ones_hbm,
                  sums_out, counts_out,
                  dstv, elist, dvlist, gbuf, onesb,
                  sums_sh, counts_sh, sem):
    cid = lax.axis_index("c")
    sid = lax.axis_index("s")
    wid = sid * NC + cid
    base = wid * EPW
    iota = lax.iota(jnp.int32, L)

    @pl.when(sid == 0)
    def _():
        pltpu.sync_copy(zc_hbm, sums_sh)
        pltpu.sync_copy(zc_hbm, counts_sh)

    pltpu.sync_copy(dst_hbm.at[pl.ds(base, EPW)], dstv)
    pltpu.sync_copy(ones_hbm, onesb)
    plsc.subcore_barrier()

    def scan_d(i, cnt):
        d = dstv[pl.ds(i * L, L)]
        m = d < 14
        plsc.store_compressed(elist.at[pl.ds(cnt, L)], iota + (base + i * L),
                              mask=m)
        plsc.store_compressed(dvlist.at[pl.ds(cnt, L)], d, mask=m)
        pc = plsc.all_reduce_population_count(m)
        return cnt + lax.squeeze(lax.slice_in_dim(pc, 0, 1), (0,))

    cnt_d = lax.fori_loop(0, NVR, scan_d, 0)
    plsc.store_scatter(elist, [cnt_d + iota], jnp.zeros((L,), jnp.int32))
    plsc.store_scatter(dvlist, [cnt_d + iota], jnp.full((L,), 14, jnp.int32))

    def acc_d(j, _):
        eidx = elist[pl.ds(j * L, L)]
        didx = dvlist[pl.ds(j * L, L)]
        pltpu.async_copy(feats_hbm.at[eidx], gbuf, sem).wait()
        pltpu.sync_copy(gbuf, sums_sh.at[didx], add=True)
        pltpu.sync_copy(onesb, counts_sh.at[didx], add=True)
        return 0

    lax.fori_loop(0, (cnt_d + L - 1) // L, acc_d, 0)
    plsc.subcore_barrier()

    @pl.when(sid == 0)
    def _():
        pltpu.sync_copy(sums_sh, sums_out.at[cid])
        pltpu.sync_copy(counts_sh, counts_out.at[cid])


_SC_KMAT_KW = dict(
    out_type=jax.ShapeDtypeStruct((NC, NNP, F), jnp.float32),
    scratch_types=[
        pltpu.VMEM((EPW,), jnp.int32),
        pltpu.VMEM((EPW,), jnp.int32),
        pltpu.VMEM((LIST,), jnp.int32),
        pltpu.VMEM((LIST,), jnp.int32),
        pltpu.VMEM((L, F), jnp.float32),
        pltpu.VMEM_SHARED((NNP, F), jnp.float32),
    ],
)


def _sc_kmat_body(src_hbm, dst_hbm, zk_hbm,
                  k_out,
                  srcv, dstv, svlist, sdlist, vb, k_sh):
    cid = lax.axis_index("c")
    sid = lax.axis_index("s")
    wid = sid * NC + cid
    base = wid * EPW
    iota = lax.iota(jnp.int32, L)

    pltpu.sync_copy(zk_hbm.at[pl.ds(sid * RPT, RPT)],
                    k_sh.at[pl.ds(sid * RPT, RPT)])
    pltpu.sync_copy(src_hbm.at[pl.ds(base, EPW)], srcv)
    pltpu.sync_copy(dst_hbm.at[pl.ds(base, EPW)], dstv)
    pltpu.sync_copy(zk_hbm.at[pl.ds(0, L)], vb)
    plsc.subcore_barrier()

    def scan_s(i, cnt):
        sv_ = srcv[pl.ds(i * L, L)]
        m = sv_ < 14
        plsc.store_compressed(svlist.at[pl.ds(cnt, L)], sv_, mask=m)
        plsc.store_compressed(sdlist.at[pl.ds(cnt, L)], dstv[pl.ds(i * L, L)],
                              mask=m)
        pc = plsc.all_reduce_population_count(m)
        return cnt + lax.squeeze(lax.slice_in_dim(pc, 0, 1), (0,))

    cnt_s = lax.fori_loop(0, NVR, scan_s, 0)
    plsc.store_scatter(svlist, [cnt_s + iota], jnp.full((L,), 14, jnp.int32))
    plsc.store_scatter(sdlist, [cnt_s + iota], jnp.zeros((L,), jnp.int32))
    ones16 = jnp.ones((L,), jnp.float32)
    zeros16 = jnp.zeros((L,), jnp.float32)

    def acc_s(j, _):
        sv = svlist[pl.ds(j * L, L)]
        dv = sdlist[pl.ds(j * L, L)]
        plsc.store_scatter(vb, [iota, sv], ones16)
        pltpu.sync_copy(vb, k_sh.at[dv], add=True)
        plsc.store_scatter(vb, [iota, sv], zeros16)
        return 0

    lax.fori_loop(0, (cnt_s + L - 1) // L, acc_s, 0)
    plsc.subcore_barrier()
    pltpu.sync_copy(k_sh.at[pl.ds(sid * RPT, RPT)],
                    k_out.at[cid, pl.ds(sid * RPT, RPT)])


def _tc_combine_body(sums_ref, counts_ref, k_ref, w_ref, b_ref,
                     gb_ref, flags_ref):
    sums = sums_ref[0] + sums_ref[1]
    counts = counts_ref[0] + counts_ref[1]
    rmask = lax.broadcasted_iota(jnp.int32, (16, 1), 0) < 14
    mean14 = jnp.where(rmask, sums / jnp.maximum(counts, 1.0), 0.0)
    kk = (k_ref[0] + k_ref[1])[:N_NODES, :16]
    cmask = lax.broadcasted_iota(jnp.int32, (N_NODES, 16), 1) < 14
    kkm = jnp.where(cmask, kk, 0.0)
    h1 = lax.dot_general(kkm, mean14, (((1,), (0,)), ((), ())),
                         preferred_element_type=jnp.float32,
                         precision=lax.Precision.HIGHEST)
    g = lax.dot_general(h1, w_ref[...], (((1,), (1,)), ((), ())),
                        preferred_element_type=jnp.float32,
                        precision=lax.Precision.HIGHEST)
    gb_ref[...] = 0.5 * g + 0.5 * b_ref[...]
    flags_ref[...] = (jnp.sum(kkm, axis=1) > 0.0).astype(jnp.int32)


_tc_combine = pl.pallas_call(
    _tc_combine_body,
    out_shape=(
        jax.ShapeDtypeStruct((N_NODES, F), jnp.float32),
        jax.ShapeDtypeStruct((N_NODES,), jnp.int32),
    ),
)


_SC_EMIT_KW = dict(
    out_type=jax.ShapeDtypeStruct((N_EDGES + 8, F), jnp.float32),
    scratch_types=[
        pltpu.VMEM((EPW,), jnp.int32),
        pltpu.VMEM((EPW,), jnp.int32),
        pltpu.VMEM((N_NODES,), jnp.int32),
        pltpu.VMEM((ELIST,), jnp.int32),
        pltpu.VMEM((ELIST,), jnp.int32),
        pltpu.VMEM((ELIST,), jnp.int32),
        pltpu.VMEM((CB, F), jnp.float32),
        pltpu.VMEM((KSB, L, F), jnp.float32),
        pltpu.VMEM((KSB, L, F), jnp.float32),
        pltpu.SemaphoreType.DMA,
        pltpu.SemaphoreType.DMA,
    ],
)


def _sc_emit_body(src_hbm, dst_hbm, gb_hbm, flags_hbm, brow_hbm, out_hbm,
             srcv, dstv, flv, el, sl, dl, bbuf, bufA, bufB, fsem, gsem):
    cid = lax.axis_index("c")
    sid = lax.axis_index("s")
    wid = sid * NC + cid
    base = wid * EPW
    iota = lax.iota(jnp.int32, L)

    pltpu.sync_copy(brow_hbm, bbuf)
    fills = []
    for k in range(NFILL):
        fills.append(pltpu.async_copy(
            bbuf, out_hbm.at[pl.ds(base + k * CB, CB)], fsem))

    pltpu.sync_copy(src_hbm.at[pl.ds(base, EPW)], srcv)
    pltpu.sync_copy(dst_hbm.at[pl.ds(base, EPW)], dstv)
    pltpu.sync_copy(flags_hbm, flv)

    def scan(i, cnt):
        s = srcv[pl.ds(i * L, L)]
        d = dstv[pl.ds(i * L, L)]
        fs = plsc.load_gather(flv, [s])
        fd = plsc.load_gather(flv, [d])
        m = (fs | fd) != 0
        inc = plsc.cumsum(jnp.where(m, 1, 0))
        pos = cnt + inc - 1
        plsc.store_scatter(el, [pos], iota + (base + i * L), mask=m)
        plsc.store_scatter(sl, [pos], s, mask=m)
        plsc.store_scatter(dl, [pos], d, mask=m)
        return cnt + _last(inc)

    cnt = lax.fori_loop(0, NVR, scan, 0)
    for t in range(KSB):
        off = cnt + t * L
        plsc.store_scatter(el, [off + iota], jnp.full((L,), N_EDGES, jnp.int32))
        plsc.store_scatter(sl, [off + iota], jnp.zeros((L,), jnp.int32))
        plsc.store_scatter(dl, [off + iota], jnp.zeros((L,), jnp.int32))

    for d_ in fills:
        d_.wait()

    SB = KSB * L

    def fix(q, _):
        jb = q * SB
        descs = []
        for t in range(KSB):
            sv = sl[pl.ds(jb + t * L, L)]
            dv = dl[pl.ds(jb + t * L, L)]
            descs.append(pltpu.async_copy(gb_hbm.at[sv], bufA.at[t], gsem))
            descs.append(pltpu.async_copy(gb_hbm.at[dv], bufB.at[t], gsem))
        for d_ in descs:
            d_.wait()
        for t in range(KSB):
            for r in range(L):
                for c in range(F // L):
                    cs = pl.ds(c * L, L)
                    bufA[t, r, cs] = bufA[t, r, cs] + bufB[t, r, cs]
            ev = el[pl.ds(jb + t * L, L)]
            pltpu.sync_copy(bufA.at[t], out_hbm.at[ev])
        return 0

    lax.fori_loop(0, (cnt + SB - 1) // SB, fix, 0)


@functools.lru_cache(maxsize=1)
def _build():
    mesh = plsc.VectorSubcoreMesh(
        core_axis_name="c", subcore_axis_name="s",
        num_cores=NC, num_subcores=NS)
    params = pltpu.CompilerParams(needs_layout_passes=False)
    sc_sums = pl.kernel(_sc_sums_body, mesh=mesh, compiler_params=params,
                        **_SC_SUMS_KW)
    sc_kmat = pl.kernel(_sc_kmat_body, mesh=mesh, compiler_params=params,
                        **_SC_KMAT_KW)
    sc_emit = pl.kernel(_sc_emit_body, mesh=mesh, compiler_params=params,
                        **_SC_EMIT_KW)
    return sc_sums, sc_kmat, sc_emit


def kernel(inputs, edge_index, W, b):
    _sc_sums, _sc_kmat, _sc_emit = _build()
    src = edge_index[0]
    dst = edge_index[1]
    zk = jnp.zeros((NNP, F), jnp.float32)
    zc = jnp.zeros((16, F), jnp.float32)
    onesc = jnp.ones((16, F), jnp.float32)
    sums_p, counts_p = _sc_sums(dst, inputs, zc, onesc)
    k_p = _sc_kmat(src, dst, zk)
    gb, flags = _tc_combine(sums_p, counts_p, k_p, W, b.reshape(1, F))
    brow = jnp.broadcast_to(b, (CB, F))
    outp = _sc_emit(src, dst, gb, flags, brow)
    return outp[:N_EDGES]


# speedup vs baseline: 1.2171x; 1.2171x over previous
import functools

import jax
import jax.numpy as jnp
from jax import lax
from jax.experimental import pallas as pl
from jax.experimental.pallas import tpu as pltpu
from jax.experimental.pallas import tpu_sc as plsc

N_NODES = 10000
N_EDGES = 320000
F = 128

NC, NS, L = 2, 16, 16
NW = NC * NS
EPW = N_EDGES // NW
NVR = EPW // L
LIST = EPW + L
NNP = 10240
RPT = NNP // NS
CB = 400
NFILL = EPW // CB

def _last(v):
    return lax.squeeze(lax.slice_in_dim(v, L - 1, L), (0,))


_SC_SUMS_KW = dict(
    out_type=(
        jax.ShapeDtypeStruct((NC, 16, F), jnp.float32),
        jax.ShapeDtypeStruct((NC, 16, F), jnp.float32),
    ),
    scratch_types=[
        pltpu.VMEM((EPW,), jnp.int32),
        pltpu.VMEM((LIST,), jnp.int32),
        pltpu.VMEM((LIST,), jnp.int32),
        pltpu.VMEM((L, F), jnp.float32),
        pltpu.VMEM((L, F), jnp.float32),
        pltpu.VMEM_SHARED((16, F), jnp.float32),
        pltpu.VMEM_SHARED((16, F), jnp.float32),
        pltpu.SemaphoreType.DMA,
    ],
)


def _sc_sums_body(dst_hbm, feats_hbm, zc_hbm, ones_hbm,
                  sums_out, counts_out,
                  dstv, elist, dvlist, gbuf, onesb,
                  sums_sh, counts_sh, sem):
    cid = lax.axis_index("c")
    sid = lax.axis_index("s")
    wid = sid * NC + cid
    base = wid * EPW
    iota = lax.iota(jnp.int32, L)

    @pl.when(sid == 0)
    def _():
        pltpu.sync_copy(zc_hbm, sums_sh)
        pltpu.sync_copy(zc_hbm, counts_sh)

    pltpu.sync_copy(dst_hbm.at[pl.ds(base, EPW)], dstv)
    pltpu.sync_copy(ones_hbm, onesb)
    plsc.subcore_barrier()

    def scan_d(i, cnt):
        d = dstv[pl.ds(i * L, L)]
        m = d < 14
        plsc.store_compressed(elist.at[pl.ds(cnt, L)], iota + (base + i * L),
                              mask=m)
        plsc.store_compressed(dvlist.at[pl.ds(cnt, L)], d, mask=m)
        pc = plsc.all_reduce_population_count(m)
        return cnt + lax.squeeze(lax.slice_in_dim(pc, 0, 1), (0,))

    cnt_d = lax.fori_loop(0, NVR, scan_d, 0)
    plsc.store_scatter(elist, [cnt_d + iota], jnp.zeros((L,), jnp.int32))
    plsc.store_scatter(dvlist, [cnt_d + iota], jnp.full((L,), 14, jnp.int32))

    def acc_d(j, _):
        eidx = elist[pl.ds(j * L, L)]
        didx = dvlist[pl.ds(j * L, L)]
        pltpu.async_copy(feats_hbm.at[eidx], gbuf, sem).wait()
        pltpu.sync_copy(gbuf, sums_sh.at[didx], add=True)
        pltpu.sync_copy(onesb, counts_sh.at[didx], add=True)
        return 0

    lax.fori_loop(0, (cnt_d + L - 1) // L, acc_d, 0)
    plsc.subcore_barrier()

    @pl.when(sid == 0)
    def _():
        pltpu.sync_copy(sums_sh, sums_out.at[cid])
        pltpu.sync_copy(counts_sh, counts_out.at[cid])


_SC_KMAT_KW = dict(
    out_type=jax.ShapeDtypeStruct((NC, NNP, F), jnp.float32),
    scratch_types=[
        pltpu.VMEM((EPW,), jnp.int32),
        pltpu.VMEM((EPW,), jnp.int32),
        pltpu.VMEM((LIST,), jnp.int32),
        pltpu.VMEM((LIST,), jnp.int32),
        pltpu.VMEM((L, F), jnp.float32),
        pltpu.VMEM_SHARED((NNP, F), jnp.float32),
    ],
)


def _sc_kmat_body(src_hbm, dst_hbm, zk_hbm,
                  k_out,
                  srcv, dstv, svlist, sdlist, vb, k_sh):
    cid = lax.axis_index("c")
    sid = lax.axis_index("s")
    wid = sid * NC + cid
    base = wid * EPW
    iota = lax.iota(jnp.int32, L)

    pltpu.sync_copy(zk_hbm.at[pl.ds(sid * RPT, RPT)],
                    k_sh.at[pl.ds(sid * RPT, RPT)])
    pltpu.sync_copy(src_hbm.at[pl.ds(base, EPW)], srcv)
    pltpu.sync_copy(dst_hbm.at[pl.ds(base, EPW)], dstv)
    pltpu.sync_copy(zk_hbm.at[pl.ds(0, L)], vb)
    plsc.subcore_barrier()

    def scan_s(i, cnt):
        sv_ = srcv[pl.ds(i * L, L)]
        m = sv_ < 14
        plsc.store_compressed(svlist.at[pl.ds(cnt, L)], sv_, mask=m)
        plsc.store_compressed(sdlist.at[pl.ds(cnt, L)], dstv[pl.ds(i * L, L)],
                              mask=m)
        pc = plsc.all_reduce_population_count(m)
        return cnt + lax.squeeze(lax.slice_in_dim(pc, 0, 1), (0,))

    cnt_s = lax.fori_loop(0, NVR, scan_s, 0)
    plsc.store_scatter(svlist, [cnt_s + iota], jnp.full((L,), 14, jnp.int32))
    plsc.store_scatter(sdlist, [cnt_s + iota], jnp.zeros((L,), jnp.int32))
    ones16 = jnp.ones((L,), jnp.float32)
    zeros16 = jnp.zeros((L,), jnp.float32)

    def acc_s(j, _):
        sv = svlist[pl.ds(j * L, L)]
        dv = sdlist[pl.ds(j * L, L)]
        plsc.store_scatter(vb, [iota, sv], ones16)
        pltpu.sync_copy(vb, k_sh.at[dv], add=True)
        plsc.store_scatter(vb, [iota, sv], zeros16)
        return 0

    lax.fori_loop(0, (cnt_s + L - 1) // L, acc_s, 0)
    plsc.subcore_barrier()
    pltpu.sync_copy(k_sh.at[pl.ds(sid * RPT, RPT)],
                    k_out.at[cid, pl.ds(sid * RPT, RPT)])


def _tc_combine_body(sums_ref, counts_ref, k_ref, w_ref, b_ref,
                     gb_ref, flags_ref):
    sums = sums_ref[0] + sums_ref[1]
    counts = counts_ref[0] + counts_ref[1]
    rmask = lax.broadcasted_iota(jnp.int32, (16, 1), 0) < 14
    mean14 = jnp.where(rmask, sums / jnp.maximum(counts, 1.0), 0.0)
    kk = (k_ref[0] + k_ref[1])[:N_NODES, :16]
    cmask = lax.broadcasted_iota(jnp.int32, (N_NODES, 16), 1) < 14
    kkm = jnp.where(cmask, kk, 0.0)
    h1 = lax.dot_general(kkm, mean14, (((1,), (0,)), ((), ())),
                         preferred_element_type=jnp.float32,
                         precision=lax.Precision.HIGHEST)
    g = lax.dot_general(h1, w_ref[...], (((1,), (1,)), ((), ())),
                        preferred_element_type=jnp.float32,
                        precision=lax.Precision.HIGHEST)
    gb_ref[...] = 0.5 * g + 0.5 * b_ref[...]
    flags_ref[...] = (jnp.sum(kkm, axis=1) > 0.0).astype(jnp.int32)


_tc_combine = pl.pallas_call(
    _tc_combine_body,
    out_shape=(
        jax.ShapeDtypeStruct((N_NODES, F), jnp.float32),
        jax.ShapeDtypeStruct((N_NODES,), jnp.int32),
    ),
)


_SC_EMIT_KW = dict(
    out_type=jax.ShapeDtypeStruct((N_EDGES + 8, F), jnp.float32),
    scratch_types=[
        pltpu.VMEM((EPW,), jnp.int32),
        pltpu.VMEM((EPW,), jnp.int32),
        pltpu.VMEM((N_NODES,), jnp.int32),
        pltpu.VMEM((LIST,), jnp.int32),
        pltpu.VMEM((LIST,), jnp.int32),
        pltpu.VMEM((LIST,), jnp.int32),
        pltpu.VMEM((CB, F), jnp.float32),
        pltpu.VMEM((L, F), jnp.float32),
        pltpu.VMEM((L, F), jnp.float32),
        pltpu.SemaphoreType.DMA,
        pltpu.SemaphoreType.DMA,
    ],
)


def _sc_emit_body(src_hbm, dst_hbm, gb_hbm, flags_hbm, brow_hbm, out_hbm,
             srcv, dstv, flv, el, sl, dl, bbuf, bufA, bufB, fsem, gsem):
    cid = lax.axis_index("c")
    sid = lax.axis_index("s")
    wid = sid * NC + cid
    base = wid * EPW
    iota = lax.iota(jnp.int32, L)

    pltpu.sync_copy(brow_hbm, bbuf)
    fills = []
    for k in range(NFILL):
        fills.append(pltpu.async_copy(
            bbuf, out_hbm.at[pl.ds(base + k * CB, CB)], fsem))

    pltpu.sync_copy(src_hbm.at[pl.ds(base, EPW)], srcv)
    pltpu.sync_copy(dst_hbm.at[pl.ds(base, EPW)], dstv)
    pltpu.sync_copy(flags_hbm, flv)

    def scan(i, cnt):
        s = srcv[pl.ds(i * L, L)]
        d = dstv[pl.ds(i * L, L)]
        fs = plsc.load_gather(flv, [s])
        fd = plsc.load_gather(flv, [d])
        m = (fs | fd) != 0
        inc = plsc.cumsum(jnp.where(m, 1, 0))
        pos = cnt + inc - 1
        plsc.store_scatter(el, [pos], iota + (base + i * L), mask=m)
        plsc.store_scatter(sl, [pos], s, mask=m)
        plsc.store_scatter(dl, [pos], d, mask=m)
        return cnt + _last(inc)

    cnt = lax.fori_loop(0, NVR, scan, 0)
    plsc.store_scatter(el, [cnt + iota], jnp.full((L,), N_EDGES, jnp.int32))
    plsc.store_scatter(sl, [cnt + iota], jnp.zeros((L,), jnp.int32))
    plsc.store_scatter(dl, [cnt + iota], jnp.zeros((L,), jnp.int32))

    for d_ in fills:
        d_.wait()

    def fix(j, _):
        sv = sl[pl.ds(j * L, L)]
        dv = dl[pl.ds(j * L, L)]
        ev = el[pl.ds(j * L, L)]
        ca = pltpu.async_copy(gb_hbm.at[sv], bufA, gsem)
        cb = pltpu.async_copy(gb_hbm.at[dv], bufB, gsem)
        ca.wait()
        cb.wait()
        for r in range(L):
            for c in range(F // L):
                sl_ = pl.ds(c * L, L)
                bufA[r, sl_] = bufA[r, sl_] + bufB[r, sl_]
        pltpu.sync_copy(bufA, out_hbm.at[ev])
        return 0

    lax.fori_loop(0, (cnt + L - 1) // L, fix, 0)


@functools.lru_cache(maxsize=1)
def _build():
    mesh = plsc.VectorSubcoreMesh(
        core_axis_name="c", subcore_axis_name="s",
        num_cores=NC, num_subcores=NS)
    params = pltpu.CompilerParams(needs_layout_passes=False)
    sc_sums = pl.kernel(_sc_sums_body, mesh=mesh, compiler_params=params,
                        **_SC_SUMS_KW)
    sc_kmat = pl.kernel(_sc_kmat_body, mesh=mesh, compiler_params=params,
                        **_SC_KMAT_KW)
    sc_emit = pl.kernel(_sc_emit_body, mesh=mesh, compiler_params=params,
                        **_SC_EMIT_KW)
    return sc_sums, sc_kmat, sc_emit


def kernel(inputs, edge_index, W, b):
    _sc_sums, _sc_kmat, _sc_emit = _build()
    src = edge_index[0]
    dst = edge_index[1]
    zk = jnp.zeros((NNP, F), jnp.float32)
    zc = jnp.zeros((16, F), jnp.float32)
    onesc = jnp.ones((16, F), jnp.float32)
    sums_p, counts_p = _sc_sums(dst, inputs, zc, onesc)
    k_p = _sc_kmat(src, dst, zk)
    gb, flags = _tc_combine(sums_p, counts_p, k_p, W, b.reshape(1, F))
    brow = jnp.broadcast_to(b, (CB, F))
    outp = _sc_emit(src, dst, gb, flags, brow)
    return outp[:N_EDGES]
